# baseline (speedup 1.0000x reference)
import functools

import jax
import jax.numpy as jnp
from jax import lax
from jax.experimental import pallas as pl
from jax.experimental.pallas import tpu as pltpu
from jax.experimental.pallas import tpu_sc as plsc

N = 10000
R = 10240
E = 320000
NC = 2
NS = 16
NW = NC * NS
CH = 80
KC = 125
RPW = R // NS
DCH = 80
DKC = RPW // DCH
RP8 = R // 8

_mesh = plsc.VectorSubcoreMesh(core_axis_name="c", subcore_axis_name="s")
_sc_params = pltpu.CompilerParams(use_tc_tiling_on_sc=False,
                                  needs_layout_passes=False)


def _fill_rows(ref, nrows, ncols, value):
    vec = jnp.full((16,), value, jnp.float32)

    def body(i, carry):
        for k in range(ncols // 16):
            ref[i, pl.ds(16 * k, 16)] = vec
        return carry

    lax.fori_loop(0, nrows, body, 0)


@functools.partial(
    pl.kernel,
    out_type=(
        jax.ShapeDtypeStruct((NC, R), jnp.float32),
        jax.ShapeDtypeStruct((NC, R), jnp.float32),
        jax.ShapeDtypeStruct((NC, RP8, 128), jnp.float32),
    ),
    mesh=_mesh,
    compiler_params=_sc_params,
    scratch_types=[
        pltpu.VMEM((KC * CH,), jnp.int32),
        pltpu.VMEM((KC * CH,), jnp.int32),
        pltpu.VMEM((CH, 16), jnp.float32),
        pltpu.VMEM((DCH, 16), jnp.float32),
        pltpu.VMEM((RPW,), jnp.float32),
        pltpu.VMEM((DCH * 16 // 128, 128), jnp.float32),
        pltpu.VMEM_SHARED((R, 16), jnp.float32),
        pltpu.VMEM_SHARED((R, 16), jnp.float32),
        pltpu.SemaphoreType.DMA,
        pltpu.SemaphoreType.DMA,
    ],
)
def _sc_degrees(edges_hbm, out_s_hbm, out_d_hbm, out_dp_hbm,
                src_v, dst_v, stage_v, ex_v, cvec, pview,
                acc_s, acc_d, sem_s, sem_d):
    cid = lax.axis_index("c")
    sid = lax.axis_index("s")
    wid = cid * NS + sid

    _fill_rows(stage_v, DCH, 16, 0.0)
    for k in range(DKC):
        r0 = sid * RPW + k * DCH
        pltpu.sync_copy(stage_v.at[pl.ds(0, DCH)], acc_s.at[pl.ds(r0, DCH)])
        pltpu.sync_copy(stage_v.at[pl.ds(0, DCH)], acc_d.at[pl.ds(r0, DCH)])
    _fill_rows(stage_v, CH, 16, 1.0)
    pltpu.sync_copy(edges_hbm.at[0, pl.ds(wid * KC * CH, KC * CH)], src_v)
    pltpu.sync_copy(edges_hbm.at[1, pl.ds(wid * KC * CH, KC * CH)], dst_v)
    plsc.subcore_barrier()

    descs = [None] * KC
    for j in range(KC):
        descs[j] = (
            pltpu.async_copy(stage_v.at[pl.ds(0, CH)], acc_s.at[src_v.at[pl.ds(j * CH, CH)]],
                             sem_s, add=True),
            pltpu.async_copy(stage_v.at[pl.ds(0, CH)], acc_d.at[dst_v.at[pl.ds(j * CH, CH)]],
                             sem_d, add=True),
        )
        if j >= 2:
            descs[j - 2][0].wait()
            descs[j - 2][1].wait()
    for j in range(max(KC - 2, 0), KC):
        descs[j][0].wait()
        descs[j][1].wait()
    plsc.subcore_barrier()

    iota = lax.iota(jnp.int32, 16)
    zcol = jnp.zeros((16,), jnp.int32)
    for acc, out_hbm, dp in ((acc_s, out_s_hbm, None), (acc_d, out_d_hbm, out_dp_hbm)):
        for k in range(DKC):
            r0 = sid * RPW + k * DCH
            pltpu.sync_copy(acc.at[pl.ds(r0, DCH)], ex_v)
            for m in range(DCH // 16):
                vals = plsc.load_gather(ex_v, [iota + 16 * m, zcol])
                cvec[pl.ds(k * DCH + 16 * m, 16)] = vals
            if dp is not None:
                for r in range(DCH):
                    pview[r // 8, pl.ds((r % 8) * 16, 16)] = ex_v[r, :]
                p0 = r0 * 16 // 128
                pltpu.sync_copy(pview, dp.at[cid, pl.ds(p0, DCH * 16 // 128)])
        pltpu.sync_copy(cvec, out_hbm.at[cid, pl.ds(sid * RPW, RPW)])


def _make_sc_agg(D, packed):
    out_shape = (NC, RP8, 128) if packed else (NC, R, D)
    pview_types = [pltpu.VMEM((DCH * D // 128, 128), jnp.float32)] if packed else []
    if packed:
        pview_types.append(pltpu.VMEM_SHARED((R, D), jnp.float32))

    @functools.partial(
        pl.kernel,
        out_type=jax.ShapeDtypeStruct(out_shape, jnp.float32),
        mesh=_mesh,
        compiler_params=_sc_params,
        scratch_types=[
            pltpu.VMEM((KC * CH,), jnp.int32),
            pltpu.VMEM((KC * CH,), jnp.int32),
            pltpu.VMEM((CH, D), jnp.float32),
            pltpu.VMEM((CH, D), jnp.float32),
            pltpu.VMEM_SHARED((R, D), jnp.float32),
            pltpu.SemaphoreType.DMA,
            pltpu.SemaphoreType.DMA,
        ] + pview_types,
    )
    def agg(table_hbm, edges_hbm, out_hbm,
            src_v, dst_v, buf0, buf1, acc, sem0, sem1, *maybe_pview):
        cid = lax.axis_index("c")
        sid = lax.axis_index("s")
        wid = cid * NS + sid
        bufs = (buf0, buf1)
        sems = (sem0, sem1)

        _fill_rows(buf0, DCH, D, 0.0)
        for k in range(DKC):
            r0 = sid * RPW + k * DCH
            pltpu.sync_copy(buf0.at[pl.ds(0, DCH)], acc.at[pl.ds(r0, DCH)])
        pltpu.sync_copy(edges_hbm.at[0, pl.ds(wid * KC * CH, KC * CH)], src_v)
        pltpu.sync_copy(edges_hbm.at[1, pl.ds(wid * KC * CH, KC * CH)], dst_v)
        if packed:
            table_sp = maybe_pview[1]
            for k in range(DKC):
                r0 = sid * RPW + k * DCH
                pltpu.sync_copy(table_hbm.at[pl.ds(r0, DCH)],
                                buf1.at[pl.ds(0, DCH)])
                pltpu.sync_copy(buf1.at[pl.ds(0, DCH)],
                                table_sp.at[pl.ds(r0, DCH)])
            table = table_sp
        else:
            table = table_hbm
        plsc.subcore_barrier()

        desc = [None] * KC
        desc[0] = pltpu.async_copy(table.at[src_v.at[pl.ds(0, CH)]], buf0, sem0)
        for j in range(KC):
            if j + 1 < KC:
                desc[j + 1] = pltpu.async_copy(
                    table.at[src_v.at[pl.ds((j + 1) * CH, CH)]], bufs[(j + 1) % 2],
                    sems[(j + 1) % 2])
            desc[j].wait()
            pltpu.sync_copy(bufs[j % 2], acc.at[dst_v.at[pl.ds(j * CH, CH)]], add=True)
        plsc.subcore_barrier()

        for k in range(DKC):
            r0 = sid * RPW + k * DCH
            pltpu.sync_copy(acc.at[pl.ds(r0, DCH)], buf0.at[pl.ds(0, DCH)])
            if packed:
                pview = maybe_pview[0]
                for r in range(DCH):
                    pview[r // 8, pl.ds((r % 8) * 16, 16)] = buf0[r, :]
                p0 = (sid * RPW + k * DCH) * D // 128
                pltpu.sync_copy(pview, out_hbm.at[cid, pl.ds(p0, DCH * D // 128)])
            else:
                pltpu.sync_copy(buf0.at[pl.ds(0, DCH)],
                                out_hbm.at[cid, pl.ds(r0, DCH)])

    return agg


_sc_agg128 = _make_sc_agg(128, packed=False)
_sc_agg16 = _make_sc_agg(16, packed=True)


BLK = 1024
BLK2 = 2048


def _norm_from(deg_ref, blk):
    deg = deg_ref[0, :] + deg_ref[1, :]
    norm = jnp.where(deg > 0, lax.rsqrt(jnp.maximum(deg, 1.0)), 0.0)
    return norm.reshape(blk, 1)


def _tc_mm_body(x_ref, w_ref, o_ref):
    o_ref[...] = jnp.dot(x_ref[...], w_ref[...],
                         preferred_element_type=jnp.float32)


def _tc_mm(x_pad, W1):
    return pl.pallas_call(
        _tc_mm_body,
        grid=(R // BLK,),
        in_specs=[
            pl.BlockSpec((BLK, 128), lambda i: (i, 0)),
            pl.BlockSpec((128, 128), lambda i: (0, 0)),
        ],
        out_specs=pl.BlockSpec((BLK, 128), lambda i: (i, 0)),
        out_shape=jax.ShapeDtypeStruct((R, 128), jnp.float32),
    )(x_pad, W1)


def _tc_scale_body(m_ref, degs_ref, o_ref):
    o_ref[...] = m_ref[...] * _norm_from(degs_ref, BLK)


def _tc_scale(mm, deg_s):
    return pl.pallas_call(
        _tc_scale_body,
        grid=(R // BLK,),
        in_specs=[
            pl.BlockSpec((BLK, 128), lambda i: (i, 0)),
            pl.BlockSpec((NC, BLK), lambda i: (0, i)),
        ],
        out_specs=pl.BlockSpec((BLK, 128), lambda i: (i, 0)),
        out_shape=jax.ShapeDtypeStruct((R, 128), jnp.float32),
    )(mm, deg_s)


def _tc_mid_body(p_ref, degs_ref, degd_ref, b1_ref, w2_ref, o_ref):
    agg = p_ref[0] + p_ref[1]
    h = jax.nn.relu(agg * _norm_from(degd_ref, BLK) + b1_ref[...])
    h2 = jnp.dot(h, w2_ref[...], preferred_element_type=jnp.float32)
    o_ref[...] = h2 * _norm_from(degs_ref, BLK)


def _tc_mid(parts1, deg_s, deg_d, b1, W2):
    return pl.pallas_call(
        _tc_mid_body,
        grid=(R // BLK,),
        in_specs=[
            pl.BlockSpec((NC, BLK, 128), lambda i: (0, i, 0)),
            pl.BlockSpec((NC, BLK), lambda i: (0, i)),
            pl.BlockSpec((NC, BLK), lambda i: (0, i)),
            pl.BlockSpec((1, 128), lambda i: (0, 0)),
            pl.BlockSpec((128, 16), lambda i: (0, 0)),
        ],
        out_specs=pl.BlockSpec((BLK, 16), lambda i: (i, 0)),
        out_shape=jax.ShapeDtypeStruct((R, 16), jnp.float32),
    )(parts1, deg_s, deg_d, b1, W2)


PBLK = BLK2 * 16 // 128


def _tc_softmax_body(p_ref, degdp_ref, b2p_ref, gmask_ref, o_ref):
    agg = p_ref[0] + p_ref[1]
    deg = degdp_ref[0] + degdp_ref[1]
    norm = jnp.where(deg > 0, lax.rsqrt(jnp.maximum(deg, 1.0)), 0.0)
    z = agg * norm + b2p_ref[...]
    ez = jnp.exp(z - jnp.max(z, axis=1, keepdims=True))
    s = jnp.dot(ez, gmask_ref[...], preferred_element_type=jnp.float32,
                precision=lax.Precision.HIGHEST)
    o_ref[...] = ez / s


def _tc_softmax(parts2, deg_dp, b2p, gmask):
    return pl.pallas_call(
        _tc_softmax_body,
        grid=(R // BLK2,),
        in_specs=[
            pl.BlockSpec((NC, PBLK, 128), lambda i: (0, i, 0)),
            pl.BlockSpec((NC, PBLK, 128), lambda i: (0, i, 0)),
            pl.BlockSpec((1, 128), lambda i: (0, 0)),
            pl.BlockSpec((128, 128), lambda i: (0, 0)),
        ],
        out_specs=pl.BlockSpec((PBLK, 128), lambda i: (i, 0)),
        out_shape=jax.ShapeDtypeStruct((RP8, 128), jnp.float32),
    )(parts2, deg_dp, b2p, gmask)


def kernel(edge_index, inputs, W1, b1, W2, b2):
    edges4 = edge_index.astype(jnp.int32)

    deg_s, deg_d, deg_dp = _sc_degrees(edges4)

    x_pad = jnp.pad(inputs, ((0, R - N), (0, 0)))
    h1p = _tc_scale(_tc_mm(x_pad, W1), deg_s)
    parts1 = _sc_agg128(h1p, edges4)
    h2p = _tc_mid(parts1, deg_s, deg_d, b1.reshape(1, 128), W2)
    parts2 = _sc_agg16(h2p, edges4)
    b2p = jnp.tile(b2.reshape(1, 16), (1, 8))
    gmask = jnp.kron(jnp.eye(8, dtype=jnp.float32),
                     jnp.ones((16, 16), jnp.float32))
    out = _tc_softmax(parts2, deg_dp, b2p, gmask)
    return out.reshape(R, 16)[:N]


# speedup vs baseline: 1.0597x; 1.0597x over previous
import functools

import jax
import jax.numpy as jnp
from jax import lax
from jax.experimental import pallas as pl
from jax.experimental.pallas import tpu as pltpu
from jax.experimental.pallas import tpu_sc as plsc

N = 10000
R = 10240
E = 320000
NC = 2
NS = 16
NW = NC * NS
CH = 112
EPW = 10000
CHUNKS = [(112 * j, 112) for j in range(89)] + [(9968, 32)]
KC = len(CHUNKS)
RPW = R // NS
DCH = 80
DKC = RPW // DCH
RP8 = R // 8

_mesh = plsc.VectorSubcoreMesh(core_axis_name="c", subcore_axis_name="s")
_sc_params = pltpu.CompilerParams(use_tc_tiling_on_sc=False,
                                  needs_layout_passes=False)


def _fill_rows(ref, nrows, ncols, value):
    vec = jnp.full((16,), value, jnp.float32)

    def body(i, carry):
        for k in range(ncols // 16):
            ref[i, pl.ds(16 * k, 16)] = vec
        return carry

    lax.fori_loop(0, nrows, body, 0)


@functools.partial(
    pl.kernel,
    out_type=(
        jax.ShapeDtypeStruct((NC, R), jnp.float32),
        jax.ShapeDtypeStruct((NC, R), jnp.float32),
        jax.ShapeDtypeStruct((NC, RP8, 128), jnp.float32),
    ),
    mesh=_mesh,
    compiler_params=_sc_params,
    scratch_types=[
        pltpu.VMEM((EPW,), jnp.int32),
        pltpu.VMEM((EPW,), jnp.int32),
        pltpu.VMEM((CH, 16), jnp.float32),
        pltpu.VMEM((DCH, 16), jnp.float32),
        pltpu.VMEM((RPW,), jnp.float32),
        pltpu.VMEM((DCH * 16 // 128, 128), jnp.float32),
        pltpu.VMEM_SHARED((R, 16), jnp.float32),
        pltpu.VMEM_SHARED((R, 16), jnp.float32),
        pltpu.SemaphoreType.DMA,
        pltpu.SemaphoreType.DMA,
    ],
)
def _sc_degrees(edges_hbm, out_s_hbm, out_d_hbm, out_dp_hbm,
                src_v, dst_v, stage_v, ex_v, cvec, pview,
                acc_s, acc_d, sem_s, sem_d):
    cid = lax.axis_index("c")
    sid = lax.axis_index("s")
    wid = cid * NS + sid

    _fill_rows(stage_v, DCH, 16, 0.0)
    for k in range(DKC):
        r0 = sid * RPW + k * DCH
        pltpu.sync_copy(stage_v.at[pl.ds(0, DCH)], acc_s.at[pl.ds(r0, DCH)])
        pltpu.sync_copy(stage_v.at[pl.ds(0, DCH)], acc_d.at[pl.ds(r0, DCH)])
    _fill_rows(stage_v, CH, 16, 1.0)
    pltpu.sync_copy(edges_hbm.at[0, pl.ds(wid * EPW, EPW)], src_v)
    pltpu.sync_copy(edges_hbm.at[1, pl.ds(wid * EPW, EPW)], dst_v)
    plsc.subcore_barrier()

    descs = [None] * KC
    for j, (off, clen) in enumerate(CHUNKS):
        descs[j] = (
            pltpu.async_copy(stage_v.at[pl.ds(0, clen)],
                             acc_s.at[src_v.at[pl.ds(off, clen)]],
                             sem_s, add=True),
            pltpu.async_copy(stage_v.at[pl.ds(0, clen)],
                             acc_d.at[dst_v.at[pl.ds(off, clen)]],
                             sem_d, add=True),
        )
        if j >= 2:
            descs[j - 2][0].wait()
            descs[j - 2][1].wait()
    for j in range(max(KC - 2, 0), KC):
        descs[j][0].wait()
        descs[j][1].wait()
    plsc.subcore_barrier()

    iota = lax.iota(jnp.int32, 16)
    zcol = jnp.zeros((16,), jnp.int32)
    for acc, out_hbm, dp in ((acc_s, out_s_hbm, None), (acc_d, out_d_hbm, out_dp_hbm)):
        for k in range(DKC):
            r0 = sid * RPW + k * DCH
            pltpu.sync_copy(acc.at[pl.ds(r0, DCH)], ex_v)
            for m in range(DCH // 16):
                vals = plsc.load_gather(ex_v, [iota + 16 * m, zcol])
                cvec[pl.ds(k * DCH + 16 * m, 16)] = vals
            if dp is not None:
                for r in range(DCH):
                    pview[r // 8, pl.ds((r % 8) * 16, 16)] = ex_v[r, :]
                p0 = r0 * 16 // 128
                pltpu.sync_copy(pview, dp.at[cid, pl.ds(p0, DCH * 16 // 128)])
        pltpu.sync_copy(cvec, out_hbm.at[cid, pl.ds(sid * RPW, RPW)])


def _make_sc_agg(D, packed):
    out_shape = (NC, RP8, 128) if packed else (NC, R, D)
    pview_types = [pltpu.VMEM((DCH * D // 128, 128), jnp.float32)] if packed else []
    if packed:
        pview_types.append(pltpu.VMEM_SHARED((R, D), jnp.float32))

    @functools.partial(
        pl.kernel,
        out_type=jax.ShapeDtypeStruct(out_shape, jnp.float32),
        mesh=_mesh,
        compiler_params=_sc_params,
        scratch_types=[
            pltpu.VMEM((EPW,), jnp.int32),
            pltpu.VMEM((EPW,), jnp.int32),
            pltpu.VMEM((CH, D), jnp.float32),
            pltpu.VMEM((CH, D), jnp.float32),
            pltpu.VMEM_SHARED((R, D), jnp.float32),
            pltpu.SemaphoreType.DMA,
            pltpu.SemaphoreType.DMA,
        ] + pview_types,
    )
    def agg(table_hbm, edges_hbm, out_hbm,
            src_v, dst_v, buf0, buf1, acc, sem0, sem1, *maybe_pview):
        cid = lax.axis_index("c")
        sid = lax.axis_index("s")
        wid = cid * NS + sid
        bufs = (buf0, buf1)
        sems = (sem0, sem1)

        _fill_rows(buf0, DCH, D, 0.0)
        for k in range(DKC):
            r0 = sid * RPW + k * DCH
            pltpu.sync_copy(buf0.at[pl.ds(0, DCH)], acc.at[pl.ds(r0, DCH)])
        pltpu.sync_copy(edges_hbm.at[0, pl.ds(wid * EPW, EPW)], src_v)
        pltpu.sync_copy(edges_hbm.at[1, pl.ds(wid * EPW, EPW)], dst_v)
        if packed:
            table_sp = maybe_pview[1]
            for k in range(DKC):
                r0 = sid * RPW + k * DCH
                pltpu.sync_copy(table_hbm.at[pl.ds(r0, DCH)],
                                buf1.at[pl.ds(0, DCH)])
                pltpu.sync_copy(buf1.at[pl.ds(0, DCH)],
                                table_sp.at[pl.ds(r0, DCH)])
            table = table_sp
        else:
            table = table_hbm
        plsc.subcore_barrier()

        desc = [None] * KC
        desc[0] = pltpu.async_copy(
            table.at[src_v.at[pl.ds(0, CHUNKS[0][1])]],
            buf0.at[pl.ds(0, CHUNKS[0][1])], sem0)
        for j, (off, clen) in enumerate(CHUNKS):
            if j + 1 < KC:
                noff, nlen = CHUNKS[j + 1]
                desc[j + 1] = pltpu.async_copy(
                    table.at[src_v.at[pl.ds(noff, nlen)]],
                    bufs[(j + 1) % 2].at[pl.ds(0, nlen)],
                    sems[(j + 1) % 2])
            desc[j].wait()
            pltpu.sync_copy(bufs[j % 2].at[pl.ds(0, clen)],
                            acc.at[dst_v.at[pl.ds(off, clen)]], add=True)
        plsc.subcore_barrier()

        for k in range(DKC):
            r0 = sid * RPW + k * DCH
            pltpu.sync_copy(acc.at[pl.ds(r0, DCH)], buf0.at[pl.ds(0, DCH)])
            if packed:
                pview = maybe_pview[0]
                for r in range(DCH):
                    pview[r // 8, pl.ds((r % 8) * 16, 16)] = buf0[r, :]
                p0 = (sid * RPW + k * DCH) * D // 128
                pltpu.sync_copy(pview, out_hbm.at[cid, pl.ds(p0, DCH * D // 128)])
            else:
                pltpu.sync_copy(buf0.at[pl.ds(0, DCH)],
                                out_hbm.at[cid, pl.ds(r0, DCH)])

    return agg


_sc_agg128 = _make_sc_agg(128, packed=False)
_sc_agg16 = _make_sc_agg(16, packed=True)


BLK = 1024
BLK2 = 2048


def _norm_from(deg_ref, blk):
    deg = deg_ref[0, :] + deg_ref[1, :]
    norm = jnp.where(deg > 0, lax.rsqrt(jnp.maximum(deg, 1.0)), 0.0)
    return norm.reshape(blk, 1)


def _tc_mm_body(x_ref, w_ref, o_ref):
    o_ref[...] = jnp.dot(x_ref[...], w_ref[...],
                         preferred_element_type=jnp.float32)


def _tc_mm(x_pad, W1):
    return pl.pallas_call(
        _tc_mm_body,
        grid=(R // BLK,),
        in_specs=[
            pl.BlockSpec((BLK, 128), lambda i: (i, 0)),
            pl.BlockSpec((128, 128), lambda i: (0, 0)),
        ],
        out_specs=pl.BlockSpec((BLK, 128), lambda i: (i, 0)),
        out_shape=jax.ShapeDtypeStruct((R, 128), jnp.float32),
    )(x_pad, W1)


def _tc_scale_body(m_ref, degs_ref, o_ref):
    o_ref[...] = m_ref[...] * _norm_from(degs_ref, BLK)


def _tc_scale(mm, deg_s):
    return pl.pallas_call(
        _tc_scale_body,
        grid=(R // BLK,),
        in_specs=[
            pl.BlockSpec((BLK, 128), lambda i: (i, 0)),
            pl.BlockSpec((NC, BLK), lambda i: (0, i)),
        ],
        out_specs=pl.BlockSpec((BLK, 128), lambda i: (i, 0)),
        out_shape=jax.ShapeDtypeStruct((R, 128), jnp.float32),
    )(mm, deg_s)


def _tc_mid_body(p_ref, degs_ref, degd_ref, b1_ref, w2_ref, o_ref):
    agg = p_ref[0] + p_ref[1]
    h = jax.nn.relu(agg * _norm_from(degd_ref, BLK) + b1_ref[...])
    h2 = jnp.dot(h, w2_ref[...], preferred_element_type=jnp.float32)
    o_ref[...] = h2 * _norm_from(degs_ref, BLK)


def _tc_mid(parts1, deg_s, deg_d, b1, W2):
    return pl.pallas_call(
        _tc_mid_body,
        grid=(R // BLK,),
        in_specs=[
            pl.BlockSpec((NC, BLK, 128), lambda i: (0, i, 0)),
            pl.BlockSpec((NC, BLK), lambda i: (0, i)),
            pl.BlockSpec((NC, BLK), lambda i: (0, i)),
            pl.BlockSpec((1, 128), lambda i: (0, 0)),
            pl.BlockSpec((128, 16), lambda i: (0, 0)),
        ],
        out_specs=pl.BlockSpec((BLK, 16), lambda i: (i, 0)),
        out_shape=jax.ShapeDtypeStruct((R, 16), jnp.float32),
    )(parts1, deg_s, deg_d, b1, W2)


PBLK = BLK2 * 16 // 128


def _tc_softmax_body(p_ref, degdp_ref, b2p_ref, gmask_ref, o_ref):
    agg = p_ref[0] + p_ref[1]
    deg = degdp_ref[0] + degdp_ref[1]
    norm = jnp.where(deg > 0, lax.rsqrt(jnp.maximum(deg, 1.0)), 0.0)
    z = agg * norm + b2p_ref[...]
    ez = jnp.exp(z - jnp.max(z, axis=1, keepdims=True))
    s = jnp.dot(ez, gmask_ref[...], preferred_element_type=jnp.float32,
                precision=lax.Precision.HIGHEST)
    o_ref[...] = ez / s


def _tc_softmax(parts2, deg_dp, b2p, gmask):
    return pl.pallas_call(
        _tc_softmax_body,
        grid=(R // BLK2,),
        in_specs=[
            pl.BlockSpec((NC, PBLK, 128), lambda i: (0, i, 0)),
            pl.BlockSpec((NC, PBLK, 128), lambda i: (0, i, 0)),
            pl.BlockSpec((1, 128), lambda i: (0, 0)),
            pl.BlockSpec((128, 128), lambda i: (0, 0)),
        ],
        out_specs=pl.BlockSpec((PBLK, 128), lambda i: (i, 0)),
        out_shape=jax.ShapeDtypeStruct((RP8, 128), jnp.float32),
    )(parts2, deg_dp, b2p, gmask)


def kernel(edge_index, inputs, W1, b1, W2, b2):
    edges4 = edge_index.astype(jnp.int32)

    deg_s, deg_d, deg_dp = _sc_degrees(edges4)

    x_pad = jnp.pad(inputs, ((0, R - N), (0, 0)))
    h1p = _tc_scale(_tc_mm(x_pad, W1), deg_s)
    parts1 = _sc_agg128(h1p, edges4)
    h2p = _tc_mid(parts1, deg_s, deg_d, b1.reshape(1, 128), W2)
    parts2 = _sc_agg16(h2p, edges4)
    b2p = jnp.tile(b2.reshape(1, 16), (1, 8))
    gmask = jnp.kron(jnp.eye(8, dtype=jnp.float32),
                     jnp.ones((16, 16), jnp.float32))
    out = _tc_softmax(parts2, deg_dp, b2p, gmask)
    return out.reshape(R, 16)[:N]


# speedup vs baseline: 1.0673x; 1.0072x over previous
import functools

import jax
import jax.numpy as jnp
from jax import lax
from jax.experimental import pallas as pl
from jax.experimental.pallas import tpu as pltpu
from jax.experimental.pallas import tpu_sc as plsc

N = 10000
R = 10240
E = 320000
NC = 2
NS = 16
NW = NC * NS
EPW = 10000
CH = 112
CHUNKS112 = [(112 * j, 112) for j in range(89)] + [(9968, 32)]
CHUNKS128 = [(128 * j, 128) for j in range(78)] + [(9984, 16)]
RPW = R // NS
DCH = 80
DKC = RPW // DCH
RP8 = R // 8

_mesh = plsc.VectorSubcoreMesh(core_axis_name="c", subcore_axis_name="s")
_sc_params = pltpu.CompilerParams(use_tc_tiling_on_sc=False,
                                  needs_layout_passes=False)


def _fill_rows(ref, nrows, ncols, value):
    vec = jnp.full((16,), value, jnp.float32)

    def body(i, carry):
        for k in range(ncols // 16):
            ref[i, pl.ds(16 * k, 16)] = vec
        return carry

    lax.fori_loop(0, nrows, body, 0)


@functools.partial(
    pl.kernel,
    out_type=(
        jax.ShapeDtypeStruct((NC, R), jnp.float32),
        jax.ShapeDtypeStruct((NC, R), jnp.float32),
        jax.ShapeDtypeStruct((NC, RP8, 128), jnp.float32),
    ),
    mesh=_mesh,
    compiler_params=_sc_params,
    scratch_types=[
        pltpu.VMEM((EPW,), jnp.int32),
        pltpu.VMEM((EPW,), jnp.int32),
        pltpu.VMEM((128, 16), jnp.float32),
        pltpu.VMEM((DCH, 16), jnp.float32),
        pltpu.VMEM((RPW,), jnp.float32),
        pltpu.VMEM((DCH * 16 // 128, 128), jnp.float32),
        pltpu.VMEM_SHARED((R, 16), jnp.float32),
        pltpu.VMEM_SHARED((R, 16), jnp.float32),
        pltpu.SemaphoreType.DMA,
        pltpu.SemaphoreType.DMA,
    ],
)
def _sc_degrees(edges_hbm, out_s_hbm, out_d_hbm, out_dp_hbm,
                src_v, dst_v, stage_v, ex_v, cvec, pview,
                acc_s, acc_d, sem_s, sem_d):
    cid = lax.axis_index("c")
    sid = lax.axis_index("s")
    wid = cid * NS + sid

    _fill_rows(stage_v, DCH, 16, 0.0)
    for k in range(DKC):
        r0 = sid * RPW + k * DCH
        pltpu.sync_copy(stage_v.at[pl.ds(0, DCH)], acc_s.at[pl.ds(r0, DCH)])
        pltpu.sync_copy(stage_v.at[pl.ds(0, DCH)], acc_d.at[pl.ds(r0, DCH)])
    _fill_rows(stage_v, 128, 16, 1.0)
    pltpu.sync_copy(edges_hbm.at[0, pl.ds(wid * EPW, EPW)], src_v)
    pltpu.sync_copy(edges_hbm.at[1, pl.ds(wid * EPW, EPW)], dst_v)
    plsc.subcore_barrier()

    descs = [None] * len(CHUNKS128)
    for j, (off, clen) in enumerate(CHUNKS128):
        descs[j] = (
            pltpu.async_copy(stage_v.at[pl.ds(0, clen)],
                             acc_s.at[src_v.at[pl.ds(off, clen)]],
                             sem_s, add=True),
            pltpu.async_copy(stage_v.at[pl.ds(0, clen)],
                             acc_d.at[dst_v.at[pl.ds(off, clen)]],
                             sem_d, add=True),
        )
        if j >= 2:
            descs[j - 2][0].wait()
            descs[j - 2][1].wait()
    for j in range(len(CHUNKS128) - 2, len(CHUNKS128)):
        descs[j][0].wait()
        descs[j][1].wait()
    plsc.subcore_barrier()

    iota = lax.iota(jnp.int32, 16)
    zcol = jnp.zeros((16,), jnp.int32)
    for acc, out_hbm, dp in ((acc_s, out_s_hbm, None), (acc_d, out_d_hbm, out_dp_hbm)):
        for k in range(DKC):
            r0 = sid * RPW + k * DCH
            pltpu.sync_copy(acc.at[pl.ds(r0, DCH)], ex_v)
            for m in range(DCH // 16):
                vals = plsc.load_gather(ex_v, [iota + 16 * m, zcol])
                cvec[pl.ds(k * DCH + 16 * m, 16)] = vals
            if dp is not None:
                for r in range(DCH):
                    pview[r // 8, pl.ds((r % 8) * 16, 16)] = ex_v[r, :]
                p0 = r0 * 16 // 128
                pltpu.sync_copy(pview, dp.at[cid, pl.ds(p0, DCH * 16 // 128)])
        pltpu.sync_copy(cvec, out_hbm.at[cid, pl.ds(sid * RPW, RPW)])


def _make_sc_agg(D, packed):
    out_shape = (NC, RP8, 128) if packed else (NC, R, D)
    pview_types = [pltpu.VMEM((DCH * D // 128, 128), jnp.float32)] if packed else []
    if packed:
        pview_types.append(pltpu.VMEM_SHARED((R, D), jnp.float32))

    @functools.partial(
        pl.kernel,
        out_type=jax.ShapeDtypeStruct(out_shape, jnp.float32),
        mesh=_mesh,
        compiler_params=_sc_params,
        scratch_types=[
            pltpu.VMEM((EPW,), jnp.int32),
            pltpu.VMEM((EPW,), jnp.int32),
            pltpu.VMEM((CH if D > 16 else 128, D), jnp.float32),
            pltpu.VMEM((CH if D > 16 else 128, D), jnp.float32),
            pltpu.VMEM_SHARED((R, D), jnp.float32),
            pltpu.SemaphoreType.DMA,
            pltpu.SemaphoreType.DMA,
        ] + pview_types,
    )
    def agg(table_hbm, edges_hbm, out_hbm,
            src_v, dst_v, buf0, buf1, acc, sem0, sem1, *maybe_pview):
        cid = lax.axis_index("c")
        sid = lax.axis_index("s")
        wid = cid * NS + sid
        bufs = (buf0, buf1)
        sems = (sem0, sem1)

        _fill_rows(buf0, DCH, D, 0.0)
        for k in range(DKC):
            r0 = sid * RPW + k * DCH
            pltpu.sync_copy(buf0.at[pl.ds(0, DCH)], acc.at[pl.ds(r0, DCH)])
        pltpu.sync_copy(edges_hbm.at[0, pl.ds(wid * EPW, EPW)], src_v)
        pltpu.sync_copy(edges_hbm.at[1, pl.ds(wid * EPW, EPW)], dst_v)
        if packed:
            table_sp = maybe_pview[1]
            for k in range(DKC):
                r0 = sid * RPW + k * DCH
                pltpu.sync_copy(table_hbm.at[pl.ds(r0, DCH)],
                                buf1.at[pl.ds(0, DCH)])
                pltpu.sync_copy(buf1.at[pl.ds(0, DCH)],
                                table_sp.at[pl.ds(r0, DCH)])
            table = table_sp
        else:
            table = table_hbm
        plsc.subcore_barrier()

        chunks = CHUNKS112 if D > 16 else CHUNKS128
        desc = [None] * len(chunks)
        desc[0] = pltpu.async_copy(
            table.at[src_v.at[pl.ds(0, chunks[0][1])]],
            buf0.at[pl.ds(0, chunks[0][1])], sem0)
        for j, (off, clen) in enumerate(chunks):
            if j + 1 < len(chunks):
                noff, nlen = chunks[j + 1]
                desc[j + 1] = pltpu.async_copy(
                    table.at[src_v.at[pl.ds(noff, nlen)]],
                    bufs[(j + 1) % 2].at[pl.ds(0, nlen)],
                    sems[(j + 1) % 2])
            desc[j].wait()
            pltpu.sync_copy(bufs[j % 2].at[pl.ds(0, clen)],
                            acc.at[dst_v.at[pl.ds(off, clen)]], add=True)
        plsc.subcore_barrier()

        for k in range(DKC):
            r0 = sid * RPW + k * DCH
            pltpu.sync_copy(acc.at[pl.ds(r0, DCH)], buf0.at[pl.ds(0, DCH)])
            if packed:
                pview = maybe_pview[0]
                for r in range(DCH):
                    pview[r // 8, pl.ds((r % 8) * 16, 16)] = buf0[r, :]
                p0 = (sid * RPW + k * DCH) * D // 128
                pltpu.sync_copy(pview, out_hbm.at[cid, pl.ds(p0, DCH * D // 128)])
            else:
                pltpu.sync_copy(buf0.at[pl.ds(0, DCH)],
                                out_hbm.at[cid, pl.ds(r0, DCH)])

    return agg


_sc_agg128 = _make_sc_agg(128, packed=False)
_sc_agg16 = _make_sc_agg(16, packed=True)


BLK = 1024
BLK2 = 2048


def _norm_from(deg_ref, blk):
    deg = deg_ref[0, :] + deg_ref[1, :]
    norm = jnp.where(deg > 0, lax.rsqrt(jnp.maximum(deg, 1.0)), 0.0)
    return norm.reshape(blk, 1)


def _tc_mm_body(x_ref, w_ref, o_ref):
    o_ref[...] = jnp.dot(x_ref[...], w_ref[...],
                         preferred_element_type=jnp.float32)


def _tc_mm(x_pad, W1):
    return pl.pallas_call(
        _tc_mm_body,
        grid=(R // BLK,),
        in_specs=[
            pl.BlockSpec((BLK, 128), lambda i: (i, 0)),
            pl.BlockSpec((128, 128), lambda i: (0, 0)),
        ],
        out_specs=pl.BlockSpec((BLK, 128), lambda i: (i, 0)),
        out_shape=jax.ShapeDtypeStruct((R, 128), jnp.float32),
    )(x_pad, W1)


def _tc_scale_body(m_ref, degs_ref, o_ref):
    o_ref[...] = m_ref[...] * _norm_from(degs_ref, BLK)


def _tc_scale(mm, deg_s):
    return pl.pallas_call(
        _tc_scale_body,
        grid=(R // BLK,),
        in_specs=[
            pl.BlockSpec((BLK, 128), lambda i: (i, 0)),
            pl.BlockSpec((NC, BLK), lambda i: (0, i)),
        ],
        out_specs=pl.BlockSpec((BLK, 128), lambda i: (i, 0)),
        out_shape=jax.ShapeDtypeStruct((R, 128), jnp.float32),
    )(mm, deg_s)


def _tc_mid_body(p_ref, degs_ref, degd_ref, b1_ref, w2_ref, o_ref):
    agg = p_ref[0] + p_ref[1]
    h = jax.nn.relu(agg * _norm_from(degd_ref, BLK) + b1_ref[...])
    h2 = jnp.dot(h, w2_ref[...], preferred_element_type=jnp.float32)
    o_ref[...] = h2 * _norm_from(degs_ref, BLK)


def _tc_mid(parts1, deg_s, deg_d, b1, W2):
    return pl.pallas_call(
        _tc_mid_body,
        grid=(R // BLK,),
        in_specs=[
            pl.BlockSpec((NC, BLK, 128), lambda i: (0, i, 0)),
            pl.BlockSpec((NC, BLK), lambda i: (0, i)),
            pl.BlockSpec((NC, BLK), lambda i: (0, i)),
            pl.BlockSpec((1, 128), lambda i: (0, 0)),
            pl.BlockSpec((128, 16), lambda i: (0, 0)),
        ],
        out_specs=pl.BlockSpec((BLK, 16), lambda i: (i, 0)),
        out_shape=jax.ShapeDtypeStruct((R, 16), jnp.float32),
    )(parts1, deg_s, deg_d, b1, W2)


PBLK = BLK2 * 16 // 128


def _tc_softmax_body(p_ref, degdp_ref, b2p_ref, gmask_ref, o_ref):
    agg = p_ref[0] + p_ref[1]
    deg = degdp_ref[0] + degdp_ref[1]
    norm = jnp.where(deg > 0, lax.rsqrt(jnp.maximum(deg, 1.0)), 0.0)
    z = agg * norm + b2p_ref[...]
    ez = jnp.exp(z - jnp.max(z, axis=1, keepdims=True))
    s = jnp.dot(ez, gmask_ref[...], preferred_element_type=jnp.float32,
                precision=lax.Precision.HIGHEST)
    o_ref[...] = ez / s


def _tc_softmax(parts2, deg_dp, b2p, gmask):
    return pl.pallas_call(
        _tc_softmax_body,
        grid=(R // BLK2,),
        in_specs=[
            pl.BlockSpec((NC, PBLK, 128), lambda i: (0, i, 0)),
            pl.BlockSpec((NC, PBLK, 128), lambda i: (0, i, 0)),
            pl.BlockSpec((1, 128), lambda i: (0, 0)),
            pl.BlockSpec((128, 128), lambda i: (0, 0)),
        ],
        out_specs=pl.BlockSpec((PBLK, 128), lambda i: (i, 0)),
        out_shape=jax.ShapeDtypeStruct((RP8, 128), jnp.float32),
    )(parts2, deg_dp, b2p, gmask)


def kernel(edge_index, inputs, W1, b1, W2, b2):
    edges4 = edge_index.astype(jnp.int32)

    deg_s, deg_d, deg_dp = _sc_degrees(edges4)

    x_pad = jnp.pad(inputs, ((0, R - N), (0, 0)))
    h1p = _tc_scale(_tc_mm(x_pad, W1), deg_s)
    parts1 = _sc_agg128(h1p, edges4)
    h2p = _tc_mid(parts1, deg_s, deg_d, b1.reshape(1, 128), W2)
    parts2 = _sc_agg16(h2p, edges4)
    b2p = jnp.tile(b2.reshape(1, 16), (1, 8))
    gmask = jnp.kron(jnp.eye(8, dtype=jnp.float32),
                     jnp.ones((16, 16), jnp.float32))
    out = _tc_softmax(parts2, deg_dp, b2p, gmask)
    return out.reshape(R, 16)[:N]
